# baseline (speedup 1.0000x reference)
import jax
import jax.numpy as jnp
from jax import lax
from jax.experimental import pallas as pl
from jax.experimental.pallas import tpu as pltpu
from jax.experimental.pallas import tpu_sc as plsc

N = 10000
E = 160000
R = 2
D = 256
DA = 16
Q = 64
NSUB = 16
CHUNK = 128
NCHUNK = 80
EPAD = NSUB * NCHUNK * CHUNK
NB = 4
NPAD = 10240
ROWS_PT = NPAD // NSUB
ZROWS = 128


def _sc_aggregate_body(t0, t1, src, dst, agg_out, cnt_out,
                       sidx, didx, rows, ones_v, zrow_v, zcnt_v,
                       agg_sh, cnt_sh, gsem, ssem, csem):
    cid = lax.axis_index("c")
    sid = lax.axis_index("s")
    coff = cid * N
    row0 = sid * ROWS_PT

    def init_ones(i, _):
        ones_v[i] = jnp.ones((16,), jnp.float32)
        return ()
    lax.fori_loop(0, CHUNK, init_ones, ())

    def init_zrow(i, _):
        r = i // (Q // 16)
        j = i % (Q // 16)
        zrow_v[r, pl.ds(j * 16, 16)] = jnp.zeros((16,), jnp.float32)
        return ()
    lax.fori_loop(0, ZROWS * (Q // 16), init_zrow, ())

    def init_zcnt(i, _):
        zcnt_v[i] = jnp.zeros((16,), jnp.float32)
        return ()
    lax.fori_loop(0, ROWS_PT, init_zcnt, ())

    pltpu.sync_copy(src.at[sid], sidx)
    pltpu.sync_copy(dst.at[sid], didx)

    def shift_row(i, _):
        for k in range(CHUNK // 16):
            s = sidx[i, pl.ds(k * 16, 16)]
            sidx[i, pl.ds(k * 16, 16)] = s + coff
        return ()
    lax.fori_loop(0, NCHUNK, shift_row, ())

    for p in range(2):
        table = (t0, t1)[p]
        for k in range(ROWS_PT // ZROWS):
            pltpu.sync_copy(zrow_v, agg_sh.at[pl.ds(row0 + k * ZROWS, ZROWS)])
        if p == 0:
            pltpu.sync_copy(zcnt_v, cnt_sh.at[pl.ds(row0, ROWS_PT)])
        plsc.subcore_barrier()

        def outer(j, _):
            gds = []
            for b in range(NB):
                i = j * NB + b

                @pl.when(j > 0)
                def _(b=b):
                    pltpu.make_async_copy(
                        table.at[pl.ds(0, CHUNK)], rows.at[b],
                        ssem.at[b]).wait()
                    if p == 0:
                        pltpu.make_async_copy(cnt_out.at[pl.ds(0, CHUNK)],
                                              ones_v, csem.at[b]).wait()
                gds.append(pltpu.async_copy(
                    table.at[sidx.at[i]], rows.at[b], gsem.at[b]))
            for b in range(NB):
                i = j * NB + b
                gds[b].wait()
                pltpu.async_copy(rows.at[b], agg_sh.at[didx.at[i]],
                                 ssem.at[b], add=True)
                if p == 0:
                    pltpu.async_copy(ones_v, cnt_sh.at[didx.at[i]],
                                     csem.at[b], add=True)
            return ()
        lax.fori_loop(0, NCHUNK // NB, outer, ())
        for b in range(NB):
            pltpu.make_async_copy(table.at[pl.ds(0, CHUNK)], rows.at[b],
                                  ssem.at[b]).wait()
            if p == 0:
                pltpu.make_async_copy(cnt_out.at[pl.ds(0, CHUNK)], ones_v,
                                      csem.at[b]).wait()
        plsc.subcore_barrier()

        pltpu.sync_copy(agg_sh.at[pl.ds(row0, ROWS_PT)],
                        agg_out.at[2 * cid + p, pl.ds(row0, ROWS_PT)])
        if p == 0:
            @pl.when(cid == 0)
            def _():
                pltpu.sync_copy(cnt_sh.at[pl.ds(row0, ROWS_PT)],
                                cnt_out.at[pl.ds(row0, ROWS_PT)])
        plsc.subcore_barrier()


def _make_sc_aggregate():
    mesh = plsc.VectorSubcoreMesh(core_axis_name="c", subcore_axis_name="s")
    return pl.kernel(
        _sc_aggregate_body,
        out_type=(
            jax.ShapeDtypeStruct((4, NPAD, Q), jnp.float32),
            jax.ShapeDtypeStruct((NPAD, DA), jnp.float32),
        ),
        mesh=mesh,
        scratch_types=[
            pltpu.VMEM((NCHUNK, CHUNK), jnp.int32),
            pltpu.VMEM((NCHUNK, CHUNK), jnp.int32),
            pltpu.VMEM((NB, CHUNK, Q), jnp.float32),
            pltpu.VMEM((CHUNK, DA), jnp.float32),
            pltpu.VMEM((ZROWS, Q), jnp.float32),
            pltpu.VMEM((ROWS_PT, DA), jnp.float32),
            pltpu.VMEM_SHARED((NPAD, Q), jnp.float32),
            pltpu.VMEM_SHARED((NPAD, DA), jnp.float32),
            pltpu.SemaphoreType.DMA((NB,)),
            pltpu.SemaphoreType.DMA((NB,)),
            pltpu.SemaphoreType.DMA((NB,)),
        ],
        compiler_params=pltpu.CompilerParams(use_tc_tiling_on_sc=False),
    )


BLK = 1000


def _bf16_dot(a, b):
    return jnp.dot(a.astype(jnp.bfloat16), b.astype(jnp.bfloat16),
                   preferred_element_type=jnp.float32)


def _layer_norm(x, gg, b):
    mu = jnp.mean(x, axis=-1, keepdims=True)
    var = jnp.mean((x - mu) ** 2, axis=-1, keepdims=True)
    return (x - mu) / jnp.sqrt(var + 1e-5) * gg + b


def _mlp_block(feat, agg_ref, cnt_ref, Wa, ba, Wb, bb, g, lb):
    inv = 1.0 / jnp.maximum(cnt_ref[...][:, 0:1], 1.0)
    Waf = Wa[...]
    ga = g[...]
    bl = lb[...]
    x = _bf16_dot(feat, Waf)
    for q in range(4):
        x += _bf16_dot(agg_ref[q] * inv, Waf[q * Q:(q + 1) * Q])
    x = jax.nn.relu(_layer_norm(x + ba[...], ga, bl))
    x = _bf16_dot(x, Wb[...]) + bb[...]
    return jax.nn.relu(_layer_norm(x, ga, bl))


def _tc_mlp_body(feat_ref, agg_ref, cnt_ref, Wa, ba, Wb, bb, g, lb, h_ref):
    h_ref[...] = _mlp_block(feat_ref[...], agg_ref, cnt_ref,
                            Wa, ba, Wb, bb, g, lb)


def _make_tc_mlp():
    full = lambda *shape: pl.BlockSpec(shape, lambda i: (0,) * len(shape))
    row_blk = pl.BlockSpec((BLK, D), lambda i: (i, 0))
    return pl.pallas_call(
        _tc_mlp_body,
        grid=(N // BLK,),
        in_specs=[
            row_blk,
            pl.BlockSpec((4, BLK, Q), lambda i: (0, i, 0)),
            pl.BlockSpec((BLK, DA), lambda i: (i, 0)),
            full(D, D), full(D), full(D, D), full(D), full(D), full(D),
        ],
        out_specs=row_blk,
        out_shape=jax.ShapeDtypeStruct((N, D), jnp.float32),
    )


def _tc_mlp_combine_body(feat_ref, agg_ref, cnt_ref, Wa, ba, Wb, bb, g, lb,
                         h0_ref, ws1_ref, ws2_ref, out_ref):
    i = pl.program_id(0)
    h1 = _mlp_block(feat_ref[...], agg_ref, cnt_ref, Wa, ba, Wb, bb, g, lb)
    hs = (h0_ref[...], h1)
    ss = []
    for r in range(R):
        t = jnp.tanh(jnp.dot(hs[r], ws1_ref[r],
                             preferred_element_type=jnp.float32))
        ss.append(jnp.dot(t, ws2_ref[r][:, None],
                          preferred_element_type=jnp.float32))
    m = jnp.maximum(ss[0], ss[1])
    e0 = jnp.exp(ss[0] - m)
    e1 = jnp.exp(ss[1] - m)
    tot = e0 + e1
    h_out = (e0 / tot) * hs[0] + (e1 / tot) * hs[1]
    blk = jnp.sum(h_out, axis=0, keepdims=True) * (1.0 / N)

    @pl.when(i == 0)
    def _():
        out_ref[...] = jnp.zeros_like(out_ref)
    out_ref[...] += blk


def _make_tc_mlp_combine():
    full = lambda *shape: pl.BlockSpec(shape, lambda i: (0,) * len(shape))
    row_blk = pl.BlockSpec((BLK, D), lambda i: (i, 0))
    return pl.pallas_call(
        _tc_mlp_combine_body,
        grid=(N // BLK,),
        in_specs=[
            row_blk,
            pl.BlockSpec((4, BLK, Q), lambda i: (0, i, 0)),
            pl.BlockSpec((BLK, DA), lambda i: (i, 0)),
            full(D, D), full(D), full(D, D), full(D), full(D), full(D),
            row_blk, full(R, D, DA), full(R, DA),
        ],
        out_specs=pl.BlockSpec((1, D), lambda i: (0, 0)),
        out_shape=jax.ShapeDtypeStruct((1, D), jnp.float32),
    )


@jax.jit
def kernel(feat, edge_index, W0_0, b0_0, W0_1, b0_1, ln_g0, ln_b0,
           W1_0, b1_0, W1_1, b1_1, ln_g1, ln_b1, ws1, ws2):
    edge_index = edge_index.astype(jnp.int32)
    t0 = jnp.concatenate([feat[:, 0:Q], feat[:, 2 * Q:3 * Q]], axis=0)
    t1 = jnp.concatenate([feat[:, Q:2 * Q], feat[:, 3 * Q:4 * Q]], axis=0)
    npad_e = EPAD - E
    pad_src = jnp.zeros((R, npad_e), jnp.int32)
    pad_dst = jnp.full((R, npad_e), NPAD - 1, jnp.int32)
    srcs = jnp.concatenate([edge_index[:, 0, :], pad_src], axis=1)
    dsts = jnp.concatenate([edge_index[:, 1, :], pad_dst], axis=1)
    eidx = jnp.stack([srcs, dsts], axis=1).reshape(R, 2, NSUB, NCHUNK, CHUNK)
    sc = _make_sc_aggregate()
    agg0, cnt0 = sc(t0, t1, eidx[0, 0], eidx[0, 1])
    agg1, cnt1 = sc(t0, t1, eidx[1, 0], eidx[1, 1])
    h0 = _make_tc_mlp()(feat, agg0, cnt0,
                        W0_0, b0_0, W0_1, b0_1, ln_g0, ln_b0)
    return _make_tc_mlp_combine()(
        feat, agg1, cnt1, W1_0, b1_0, W1_1, b1_1, ln_g1, ln_b1,
        h0, ws1, ws2.reshape(R, DA))


# speedup vs baseline: 2.0093x; 2.0093x over previous
import jax
import jax.numpy as jnp
from jax import lax
from jax.experimental import pallas as pl
from jax.experimental.pallas import tpu as pltpu
from jax.experimental.pallas import tpu_sc as plsc

N = 10000
E = 160000
R = 2
D = 256
DA = 16
Q = 64
NSUB = 16
EPT = E // NSUB
CHUNK = 125
NCHUNK = EPT // CHUNK
NB = 4
NPAD = 10240
ROWS_PT = NPAD // NSUB
ZROWS = 128


def _sc_aggregate_body(t0, t1, src, dst, agg_out, cnt_out,
                       sidx, didx, rows, ones_v, zrow_v, zcnt_v,
                       agg_sh, cnt_sh, gsem, ssem, csem):
    cid = lax.axis_index("c")
    sid = lax.axis_index("s")
    coff = cid * N
    row0 = sid * ROWS_PT

    def init_ones(i, _):
        ones_v[i] = jnp.ones((16,), jnp.float32)
        return ()
    lax.fori_loop(0, CHUNK, init_ones, ())

    def init_zrow(i, _):
        r = i // (Q // 16)
        j = i % (Q // 16)
        zrow_v[r, pl.ds(j * 16, 16)] = jnp.zeros((16,), jnp.float32)
        return ()
    lax.fori_loop(0, ZROWS * (Q // 16), init_zrow, ())

    def init_zcnt(i, _):
        zcnt_v[i] = jnp.zeros((16,), jnp.float32)
        return ()
    lax.fori_loop(0, ROWS_PT, init_zcnt, ())

    pltpu.sync_copy(src.at[sid], sidx)
    pltpu.sync_copy(dst.at[sid], didx)

    def shift_row2(i, _):
        nfull = CHUNK // 16
        for k in range(nfull):
            s = sidx[i, pl.ds(k * 16, 16)]
            sidx[i, pl.ds(k * 16, 16)] = s + coff
        tail = CHUNK - nfull * 16
        s = sidx[i, pl.ds(nfull * 16 - (16 - tail), 16)]
        mask = lax.iota(jnp.int32, 16) >= (16 - tail)
        sidx[i, pl.ds(nfull * 16 - (16 - tail), 16)] = jnp.where(
            mask, s + coff, s)
        return ()
    lax.fori_loop(0, NCHUNK, shift_row2, ())

    for p in range(2):
        table = (t0, t1)[p]
        for k in range(ROWS_PT // ZROWS):
            pltpu.sync_copy(zrow_v, agg_sh.at[pl.ds(row0 + k * ZROWS, ZROWS)])
        if p == 0:
            pltpu.sync_copy(zcnt_v, cnt_sh.at[pl.ds(row0, ROWS_PT)])
        plsc.subcore_barrier()

        def outer(j, _):
            gds = []
            for b in range(NB):
                i = j * NB + b

                @pl.when(j > 0)
                def _(b=b):
                    pltpu.make_async_copy(
                        table.at[pl.ds(0, CHUNK)], rows.at[b],
                        ssem.at[b]).wait()
                    if p == 0:
                        pltpu.make_async_copy(cnt_out.at[pl.ds(0, CHUNK)],
                                              ones_v, csem.at[b]).wait()
                gds.append(pltpu.async_copy(
                    table.at[sidx.at[i]], rows.at[b], gsem.at[b]))
            for b in range(NB):
                i = j * NB + b
                gds[b].wait()
                pltpu.async_copy(rows.at[b], agg_sh.at[didx.at[i]],
                                 ssem.at[b], add=True)
                if p == 0:
                    pltpu.async_copy(ones_v, cnt_sh.at[didx.at[i]],
                                     csem.at[b], add=True)
            return ()
        lax.fori_loop(0, NCHUNK // NB, outer, ())
        for b in range(NB):
            pltpu.make_async_copy(table.at[pl.ds(0, CHUNK)], rows.at[b],
                                  ssem.at[b]).wait()
            if p == 0:
                pltpu.make_async_copy(cnt_out.at[pl.ds(0, CHUNK)], ones_v,
                                      csem.at[b]).wait()
        plsc.subcore_barrier()

        pltpu.sync_copy(agg_sh.at[pl.ds(row0, ROWS_PT)],
                        agg_out.at[2 * cid + p, pl.ds(row0, ROWS_PT)])
        if p == 0:
            @pl.when(cid == 0)
            def _():
                pltpu.sync_copy(cnt_sh.at[pl.ds(row0, ROWS_PT)],
                                cnt_out.at[pl.ds(row0, ROWS_PT)])
        plsc.subcore_barrier()


def _make_sc_aggregate():
    mesh = plsc.VectorSubcoreMesh(core_axis_name="c", subcore_axis_name="s")
    return pl.kernel(
        _sc_aggregate_body,
        out_type=(
            jax.ShapeDtypeStruct((4, NPAD, Q), jnp.float32),
            jax.ShapeDtypeStruct((NPAD, DA), jnp.float32),
        ),
        mesh=mesh,
        scratch_types=[
            pltpu.VMEM((NCHUNK, CHUNK), jnp.int32),
            pltpu.VMEM((NCHUNK, CHUNK), jnp.int32),
            pltpu.VMEM((NB, CHUNK, Q), jnp.float32),
            pltpu.VMEM((CHUNK, DA), jnp.float32),
            pltpu.VMEM((ZROWS, Q), jnp.float32),
            pltpu.VMEM((ROWS_PT, DA), jnp.float32),
            pltpu.VMEM_SHARED((NPAD, Q), jnp.float32),
            pltpu.VMEM_SHARED((NPAD, DA), jnp.float32),
            pltpu.SemaphoreType.DMA((NB,)),
            pltpu.SemaphoreType.DMA((NB,)),
            pltpu.SemaphoreType.DMA((NB,)),
        ],
        compiler_params=pltpu.CompilerParams(use_tc_tiling_on_sc=False),
    )


BLK = 2000


def _bf16_dot(a, b):
    return jnp.dot(a.astype(jnp.bfloat16), b.astype(jnp.bfloat16),
                   preferred_element_type=jnp.float32)


def _layer_norm(x, gg, b):
    mu = jnp.mean(x, axis=-1, keepdims=True)
    var = jnp.mean((x - mu) ** 2, axis=-1, keepdims=True)
    return (x - mu) / jnp.sqrt(var + 1e-5) * gg + b


def _mlp_block(feat, agg_ref, cnt_ref, Wa, ba, Wb, bb, g, lb):
    inv = 1.0 / jnp.maximum(cnt_ref[...][:, 0:1], 1.0)
    Waf = Wa[...]
    ga = g[...]
    bl = lb[...]
    x = _bf16_dot(feat, Waf)
    for q in range(4):
        x += _bf16_dot(agg_ref[q] * inv, Waf[q * Q:(q + 1) * Q])
    x = jax.nn.relu(_layer_norm(x + ba[...], ga, bl))
    x = _bf16_dot(x, Wb[...]) + bb[...]
    return jax.nn.relu(_layer_norm(x, ga, bl))


def _tc_mlp_body(feat_ref, agg_ref, cnt_ref, Wa, ba, Wb, bb, g, lb, h_ref):
    h_ref[...] = _mlp_block(feat_ref[...], agg_ref, cnt_ref,
                            Wa, ba, Wb, bb, g, lb)


def _make_tc_mlp():
    full = lambda *shape: pl.BlockSpec(shape, lambda i: (0,) * len(shape))
    row_blk = pl.BlockSpec((BLK, D), lambda i: (i, 0))
    return pl.pallas_call(
        _tc_mlp_body,
        grid=(N // BLK,),
        in_specs=[
            row_blk,
            pl.BlockSpec((4, BLK, Q), lambda i: (0, i, 0)),
            pl.BlockSpec((BLK, DA), lambda i: (i, 0)),
            full(D, D), full(D), full(D, D), full(D), full(D), full(D),
        ],
        out_specs=row_blk,
        out_shape=jax.ShapeDtypeStruct((N, D), jnp.float32),
    )


def _tc_mlp_combine_body(feat_ref, agg_ref, cnt_ref, Wa, ba, Wb, bb, g, lb,
                         h0_ref, ws1_ref, ws2_ref, out_ref):
    i = pl.program_id(0)
    h1 = _mlp_block(feat_ref[...], agg_ref, cnt_ref, Wa, ba, Wb, bb, g, lb)
    hs = (h0_ref[...], h1)
    ss = []
    for r in range(R):
        t = jnp.tanh(jnp.dot(hs[r], ws1_ref[r],
                             preferred_element_type=jnp.float32))
        ss.append(jnp.dot(t, ws2_ref[r][:, None],
                          preferred_element_type=jnp.float32))
    m = jnp.maximum(ss[0], ss[1])
    e0 = jnp.exp(ss[0] - m)
    e1 = jnp.exp(ss[1] - m)
    tot = e0 + e1
    h_out = (e0 / tot) * hs[0] + (e1 / tot) * hs[1]
    blk = jnp.sum(h_out, axis=0, keepdims=True) * (1.0 / N)

    @pl.when(i == 0)
    def _():
        out_ref[...] = jnp.zeros_like(out_ref)
    out_ref[...] += blk


def _make_tc_mlp_combine():
    full = lambda *shape: pl.BlockSpec(shape, lambda i: (0,) * len(shape))
    row_blk = pl.BlockSpec((BLK, D), lambda i: (i, 0))
    return pl.pallas_call(
        _tc_mlp_combine_body,
        grid=(N // BLK,),
        in_specs=[
            row_blk,
            pl.BlockSpec((4, BLK, Q), lambda i: (0, i, 0)),
            pl.BlockSpec((BLK, DA), lambda i: (i, 0)),
            full(D, D), full(D), full(D, D), full(D), full(D), full(D),
            row_blk, full(R, D, DA), full(R, DA),
        ],
        out_specs=pl.BlockSpec((1, D), lambda i: (0, 0)),
        out_shape=jax.ShapeDtypeStruct((1, D), jnp.float32),
    )


@jax.jit
def kernel(feat, edge_index, W0_0, b0_0, W0_1, b0_1, ln_g0, ln_b0,
           W1_0, b1_0, W1_1, b1_1, ln_g1, ln_b1, ws1, ws2):
    edge_index = edge_index.astype(jnp.int32)
    t0 = jnp.concatenate([feat[:, 0:Q], feat[:, 2 * Q:3 * Q]], axis=0)
    t1 = jnp.concatenate([feat[:, Q:2 * Q], feat[:, 3 * Q:4 * Q]], axis=0)
    eidx = edge_index.reshape(R, 2, NSUB, NCHUNK, CHUNK)
    sc = _make_sc_aggregate()
    agg0, cnt0 = sc(t0, t1, eidx[0, 0], eidx[0, 1])
    agg1, cnt1 = sc(t0, t1, eidx[1, 0], eidx[1, 1])
    h0 = _make_tc_mlp()(feat, agg0, cnt0,
                        W0_0, b0_0, W0_1, b0_1, ln_g0, ln_b0)
    return _make_tc_mlp_combine()(
        feat, agg1, cnt1, W1_0, b1_0, W1_1, b1_1, ln_g1, ln_b1,
        h0, ws1, ws2.reshape(R, DA))
